# baseline (speedup 1.0000x reference)
import functools

import jax
import jax.numpy as jnp
from jax import lax
from jax.experimental import pallas as pl
from jax.experimental.pallas import tpu as pltpu
from jax.experimental.pallas import tpu_sc as plsc

NP = 10000
EP = 160000
NL = 2000
EL = 32000
D = 128
LAYERS = 3

NPA = 10240
NLA = 2048
EPA = 163840
ELA = 32768
W = 144

NC, NS = 2, 16
NW = NC * NS
CHUNK = 128
EP_CHUNKS = EPA // (NW * CHUNK)
EL_CHUNKS = ELA // (NW * CHUNK)
ROWS_P = NPA // NS
ROWS_L = NLA // NS

f32 = jnp.float32
bf16 = jnp.bfloat16


@functools.lru_cache(maxsize=None)
def _sc_mesh():
    return plsc.VectorSubcoreMesh(core_axis_name="c", subcore_axis_name="s",
                                  num_cores=NC, num_subcores=NS)


def _make_gather_body(cpw):
    def body_fn(rd_hbm, rs_hbm, dsti_hbm, srci_hbm, ed_hbm, es_hbm,
                idx_da, idx_sa, buf_d0, buf_s0, buf_d1, buf_s1,
                gsem0, gsem1, wsem0, wsem1):
        wid = lax.axis_index("s") * NC + lax.axis_index("c")
        base = wid * cpw
        bufs_d = (buf_d0, buf_d1)
        bufs_s = (buf_s0, buf_s1)
        gsems = (gsem0, gsem1)
        wsems = (wsem0, wsem1)
        pltpu.sync_copy(dsti_hbm.at[pl.ds(base, cpw)], idx_da)
        pltpu.sync_copy(srci_hbm.at[pl.ds(base, cpw)], idx_sa)

        def gstart(j, b):
            pltpu.async_copy(rd_hbm.at[idx_da.at[j]], bufs_d[b], gsems[b])
            pltpu.async_copy(rs_hbm.at[idx_sa.at[j]], bufs_s[b], gsems[b])

        def gwait(b):
            pltpu.make_async_copy(rd_hbm.at[idx_da.at[0]], bufs_d[b],
                                  gsems[b]).wait()
            pltpu.make_async_copy(rs_hbm.at[idx_sa.at[0]], bufs_s[b],
                                  gsems[b]).wait()

        def wstart(j, b):
            r = base + j
            pltpu.async_copy(bufs_d[b], ed_hbm.at[pl.ds(r * CHUNK, CHUNK)],
                             wsems[b])
            pltpu.async_copy(bufs_s[b], es_hbm.at[pl.ds(r * CHUNK, CHUNK)],
                             wsems[b])

        def wwait(b):
            pltpu.make_async_copy(bufs_d[b], ed_hbm.at[pl.ds(0, CHUNK)],
                                  wsems[b]).wait()
            pltpu.make_async_copy(bufs_s[b], es_hbm.at[pl.ds(0, CHUNK)],
                                  wsems[b]).wait()

        gstart(0, 0)
        gstart(1, 1)

        def outer(t, carry):
            for b in (0, 1):
                j = 2 * t + b
                gwait(b)
                wstart(j, b)
                wwait(b)
                gstart(j + 2, b)
            return carry

        lax.fori_loop(0, cpw // 2 - 1, outer, 0)
        for b in (0, 1):
            gwait(b)
            wstart(cpw - 2 + b, b)
            wwait(b)

    return body_fn


@functools.lru_cache(maxsize=None)
def _sc_gather_call(n_edges):
    cpw = n_edges // (NW * CHUNK)
    return pl.kernel(
        _make_gather_body(cpw),
        out_type=[jax.ShapeDtypeStruct((n_edges, W), f32),
                  jax.ShapeDtypeStruct((n_edges, W), f32)],
        mesh=_sc_mesh(),
        compiler_params=pltpu.CompilerParams(use_tc_tiling_on_sc=False),
        scratch_types=[
            pltpu.VMEM((cpw, CHUNK), jnp.int32),
            pltpu.VMEM((cpw, CHUNK), jnp.int32),
            pltpu.VMEM((CHUNK, W), f32),
            pltpu.VMEM((CHUNK, W), f32),
            pltpu.VMEM((CHUNK, W), f32),
            pltpu.VMEM((CHUNK, W), f32),
            pltpu.SemaphoreType.DMA,
            pltpu.SemaphoreType.DMA,
            pltpu.SemaphoreType.DMA,
            pltpu.SemaphoreType.DMA,
        ],
    )


def _sc_gather(rd, rs, dstp, srcp):
    return _sc_gather_call(dstp.shape[0] * CHUNK)(rd, rs, dstp, srcp)


def _make_scatter_body(cpw):
    def body_fn(em_hbm, dsti_hbm, zeros_hbm, out_hbm, idx0, idx1,
                buf0, buf1, acc, lsem0, lsem1, ssem0, ssem1):
        c = lax.axis_index("c")
        s = lax.axis_index("s")
        wid = s * NC + c
        base = wid * cpw
        idxs = (idx0, idx1)
        bufs = (buf0, buf1)
        lsems = (lsem0, lsem1)
        ssems = (ssem0, ssem1)
        pltpu.sync_copy(zeros_hbm.at[pl.ds(s * ROWS_P, ROWS_P)],
                        acc.at[pl.ds(s * ROWS_P, ROWS_P)])
        plsc.subcore_barrier()

        def lstart(j, b):
            r = base + j
            pltpu.async_copy(dsti_hbm.at[r], idxs[b], lsems[b])
            pltpu.async_copy(em_hbm.at[pl.ds(r * CHUNK, CHUNK)], bufs[b],
                             lsems[b])

        def lwait(b):
            pltpu.make_async_copy(dsti_hbm.at[0], idxs[b], lsems[b]).wait()
            pltpu.make_async_copy(em_hbm.at[pl.ds(0, CHUNK)], bufs[b],
                                  lsems[b]).wait()

        def sstart(j, b):
            pltpu.async_copy(bufs[b], acc.at[idxs[b]], ssems[b], add=True)

        def swait(b):
            pltpu.make_async_copy(bufs[b], acc.at[idxs[b]], ssems[b]).wait()

        lstart(0, 0)
        lstart(1, 1)

        def outer(t, carry):
            for b in (0, 1):
                j = 2 * t + b
                lwait(b)
                sstart(j, b)
                swait(b)
                lstart(j + 2, b)
            return carry

        lax.fori_loop(0, cpw // 2 - 1, outer, 0)
        for b in (0, 1):
            lwait(b)
            sstart(cpw - 2 + b, b)
            swait(b)
        plsc.subcore_barrier()
        pltpu.sync_copy(acc.at[pl.ds(s * ROWS_P, ROWS_P)],
                        out_hbm.at[c, pl.ds(s * ROWS_P, ROWS_P)])

    return body_fn


@functools.lru_cache(maxsize=None)
def _sc_scatter_call(n_edges):
    cpw = n_edges // (NW * CHUNK)
    return pl.kernel(
        _make_scatter_body(cpw),
        out_type=jax.ShapeDtypeStruct((NC, NPA, W), f32),
        mesh=_sc_mesh(),
        compiler_params=pltpu.CompilerParams(use_tc_tiling_on_sc=False),
        scratch_types=[
            pltpu.VMEM((CHUNK,), jnp.int32),
            pltpu.VMEM((CHUNK,), jnp.int32),
            pltpu.VMEM((CHUNK, W), f32),
            pltpu.VMEM((CHUNK, W), f32),
            pltpu.VMEM_SHARED((NPA, W), f32),
            pltpu.SemaphoreType.DMA,
            pltpu.SemaphoreType.DMA,
            pltpu.SemaphoreType.DMA,
            pltpu.SemaphoreType.DMA,
        ],
    )


def _sc_scatter(em, dstp, zeros_p):
    return _sc_scatter_call(dstp.shape[0] * CHUNK)(em, dstp, zeros_p)


def _gin_body(hl_hbm, srci_hbm, dsti_hbm, zeros_hbm, out_hbm,
              idx_sa, idx_da, buf0, buf1, acc, gsem0, gsem1, ssem0, ssem1):
    c = lax.axis_index("c")
    s = lax.axis_index("s")
    wid = s * NC + c
    base = wid * EL_CHUNKS
    bufs = (buf0, buf1)
    gsems = (gsem0, gsem1)
    ssems = (ssem0, ssem1)
    pltpu.sync_copy(zeros_hbm.at[pl.ds(s * ROWS_L, ROWS_L)],
                    acc.at[pl.ds(s * ROWS_L, ROWS_L)])
    pltpu.sync_copy(srci_hbm.at[pl.ds(base, EL_CHUNKS)], idx_sa)
    pltpu.sync_copy(dsti_hbm.at[pl.ds(base, EL_CHUNKS)], idx_da)
    plsc.subcore_barrier()

    def gstart(j, b):
        pltpu.async_copy(hl_hbm.at[idx_sa.at[j]], bufs[b], gsems[b])

    def gwait(b):
        pltpu.make_async_copy(hl_hbm.at[idx_sa.at[0]], bufs[b],
                              gsems[b]).wait()

    def sstart(j, b):
        pltpu.async_copy(bufs[b], acc.at[idx_da.at[j]], ssems[b], add=True)

    def swait(b):
        pltpu.make_async_copy(bufs[b], acc.at[idx_da.at[0]], ssems[b]).wait()

    gstart(0, 0)
    gstart(1, 1)

    def outer(t, carry):
        for b in (0, 1):
            j = 2 * t + b
            gwait(b)
            sstart(j, b)
            swait(b)
            gstart(j + 2, b)
        return carry

    lax.fori_loop(0, EL_CHUNKS // 2 - 1, outer, 0)
    for b in (0, 1):
        gwait(b)
        sstart(EL_CHUNKS - 2 + b, b)
        swait(b)
    plsc.subcore_barrier()
    pltpu.sync_copy(acc.at[pl.ds(s * ROWS_L, ROWS_L)],
                    out_hbm.at[c, pl.ds(s * ROWS_L, ROWS_L)])


@functools.lru_cache(maxsize=None)
def _sc_gin_agg_call():
    return pl.kernel(
        _gin_body,
        out_type=jax.ShapeDtypeStruct((NC, NLA, D), f32),
        mesh=_sc_mesh(),
        compiler_params=pltpu.CompilerParams(use_tc_tiling_on_sc=False),
        scratch_types=[
            pltpu.VMEM((EL_CHUNKS, CHUNK), jnp.int32),
            pltpu.VMEM((EL_CHUNKS, CHUNK), jnp.int32),
            pltpu.VMEM((CHUNK, D), f32),
            pltpu.VMEM((CHUNK, D), f32),
            pltpu.VMEM_SHARED((NLA, D), f32),
            pltpu.SemaphoreType.DMA,
            pltpu.SemaphoreType.DMA,
            pltpu.SemaphoreType.DMA,
            pltpu.SemaphoreType.DMA,
        ],
    )


def _sc_gin_agg(h_l, srcl, dstl, zeros_l):
    return _sc_gin_agg_call()(h_l, srcl, dstl, zeros_l)


BP = 512
BE = 1024


def _k1_body(h_ref, c_ref, wa_ref, wb_ref, be1_ref, rd_ref, rs_ref):
    h = h_ref[...]
    a = jnp.dot(h, wa_ref[...], preferred_element_type=f32) + be1_ref[...]
    b = jnp.dot(h, wb_ref[...], preferred_element_type=f32)
    cpad = c_ref[...]
    rd_ref[:, :D] = a
    rd_ref[:, D:] = cpad
    rs_ref[:, :D] = b
    rs_ref[:, D:] = cpad


def _tc_node_project(h, c16, wa, wb, be1):
    grid = NPA // BP
    return pl.pallas_call(
        _k1_body,
        grid=(grid,),
        in_specs=[
            pl.BlockSpec((BP, D), lambda i: (i, 0)),
            pl.BlockSpec((BP, 16), lambda i: (i, 0)),
            pl.BlockSpec((D, D), lambda i: (0, 0)),
            pl.BlockSpec((D, D), lambda i: (0, 0)),
            pl.BlockSpec((1, D), lambda i: (0, 0)),
        ],
        out_specs=[pl.BlockSpec((BP, W), lambda i: (i, 0)),
                   pl.BlockSpec((BP, W), lambda i: (i, 0))],
        out_shape=[jax.ShapeDtypeStruct((NPA, W), f32),
                   jax.ShapeDtypeStruct((NPA, W), f32)],
    )(h, c16, wa, wb, be1)


def _k3_body(ed_ref, es_ref, wce_ref, we2_ref, be2_ref, wc_ref, em_ref):
    ed = ed_ref[...]
    es = es_ref[...]
    a = ed[:, :D]
    b = es[:, :D]
    rel = ed[:, D:] - es[:, D:]
    d2 = jnp.sum(rel * rel, axis=1, keepdims=True)
    pre = a + b + d2 * wce_ref[...]
    m1 = jax.nn.silu(pre)
    m2 = jax.nn.silu(jnp.dot(m1, we2_ref[...], preferred_element_type=f32)
                     + be2_ref[...])
    cw = jnp.sum(m2 * wc_ref[...], axis=1, keepdims=True)
    relcw = rel * cw
    lane = lax.broadcasted_iota(jnp.int32, relcw.shape, 1)
    relcw = jnp.where(lane == 3, 1.0, relcw)
    em_ref[:, :D] = m2
    em_ref[:, D:] = relcw


def _tc_edge_mlp(ed, es, wce, we2, be2, wc, off, n):
    grid = n // BE
    o = off // BE
    return pl.pallas_call(
        _k3_body,
        grid=(grid,),
        in_specs=[
            pl.BlockSpec((BE, W), lambda i: (i + o, 0)),
            pl.BlockSpec((BE, W), lambda i: (i + o, 0)),
            pl.BlockSpec((1, D), lambda i: (0, 0)),
            pl.BlockSpec((D, D), lambda i: (0, 0)),
            pl.BlockSpec((1, D), lambda i: (0, 0)),
            pl.BlockSpec((1, D), lambda i: (0, 0)),
        ],
        out_specs=pl.BlockSpec((BE, W), lambda i: (i, 0)),
        out_shape=jax.ShapeDtypeStruct((n, W), f32),
    )(ed, es, wce, we2, be2, wc)


def _k5_body(h_ref, c_ref, sa_ref, sb_ref,
             wn1a_ref, wn1b_ref, bn1_ref,
             wn2_ref, bn2_ref, wq_ref, wk_ref, wv_ref,
             hout_ref, cout_ref, q_ref, k_ref, v_ref):
    S = (sa_ref[0] + sa_ref[1]) + (sb_ref[0] + sb_ref[1])
    agg = S[:, :D]
    sc = S[:, D:]
    deg = jnp.maximum(sc[:, 3:4], 1.0)
    lane = lax.broadcasted_iota(jnp.int32, sc.shape, 1)
    cout_ref[...] = c_ref[...] + jnp.where(lane < 3, sc / deg, 0.0)
    h = h_ref[...]
    t = jax.nn.silu(jnp.dot(h, wn1a_ref[...], preferred_element_type=f32)
                    + jnp.dot(agg, wn1b_ref[...], preferred_element_type=f32)
                    + bn1_ref[...])
    hn = h + jnp.dot(t, wn2_ref[...], preferred_element_type=f32) + bn2_ref[...]
    hout_ref[...] = hn
    q_ref[...] = jnp.dot(hn, wq_ref[...], preferred_element_type=f32)
    k_ref[...] = jnp.dot(hn, wk_ref[...], preferred_element_type=f32)
    v_ref[...] = jnp.dot(hn, wv_ref[...], preferred_element_type=f32)


def _tc_node_update(h, c16, sa, sb, wn1a, wn1b, bn1, wn2, bn2,
                    wq, wk, wv):
    grid = NPA // BP
    wspec = pl.BlockSpec((D, D), lambda i: (0, 0))
    rowspec = pl.BlockSpec((BP, D), lambda i: (i, 0))
    return pl.pallas_call(
        _k5_body,
        grid=(grid,),
        in_specs=[
            rowspec,
            pl.BlockSpec((BP, 16), lambda i: (i, 0)),
            pl.BlockSpec((NC, BP, W), lambda i: (0, i, 0)),
            pl.BlockSpec((NC, BP, W), lambda i: (0, i, 0)),
            wspec,
            wspec,
            pl.BlockSpec((1, D), lambda i: (0, 0)),
            wspec,
            pl.BlockSpec((1, D), lambda i: (0, 0)),
            wspec,
            wspec,
            wspec,
        ],
        out_specs=[rowspec,
                   pl.BlockSpec((BP, 16), lambda i: (i, 0)),
                   rowspec, rowspec, rowspec],
        out_shape=[jax.ShapeDtypeStruct((NPA, D), f32),
                   jax.ShapeDtypeStruct((NPA, 16), f32),
                   jax.ShapeDtypeStruct((NPA, D), f32),
                   jax.ShapeDtypeStruct((NPA, D), f32),
                   jax.ShapeDtypeStruct((NPA, D), f32)],
    )(h, c16, sa, sb, wn1a, wn1b, bn1, wn2, bn2, wq, wk, wv)


def _k7_body(h_ref, a2_ref, eps_ref, wg1_ref, bg1_ref, wg2_ref,
             bg2_ref, wq_ref, wk_ref, wv_ref, out_ref, q_ref, k_ref, v_ref):
    z = (1.0 + eps_ref[0]) * h_ref[...] + a2_ref[0] + a2_ref[1]
    h1 = jax.nn.relu(jnp.dot(z, wg1_ref[...], preferred_element_type=f32)
                     + bg1_ref[...])
    hn = jnp.dot(h1, wg2_ref[...], preferred_element_type=f32) + bg2_ref[...]
    out_ref[...] = hn
    q_ref[...] = jnp.dot(hn, wq_ref[...], preferred_element_type=f32)
    k_ref[...] = jnp.dot(hn, wk_ref[...], preferred_element_type=f32)
    v_ref[...] = jnp.dot(hn, wv_ref[...], preferred_element_type=f32)


def _tc_gin_update(h, a2, eps_i, wg1, bg1, wg2, bg2, wq, wk, wv):
    grid = NLA // BP
    wspec = pl.BlockSpec((D, D), lambda i: (0, 0))
    rowspec = pl.BlockSpec((BP, D), lambda i: (i, 0))
    return pl.pallas_call(
        _k7_body,
        grid=(grid,),
        in_specs=[
            rowspec,
            pl.BlockSpec((NC, BP, D), lambda i: (0, i, 0)),
            pl.BlockSpec(memory_space=pltpu.SMEM),
            wspec,
            pl.BlockSpec((1, D), lambda i: (0, 0)),
            wspec,
            pl.BlockSpec((1, D), lambda i: (0, 0)),
            wspec,
            wspec,
            wspec,
        ],
        out_specs=[rowspec] * 4,
        out_shape=[jax.ShapeDtypeStruct((NLA, D), f32)] * 4,
    )(h, a2, eps_i, wg1, bg1, wg2, bg2, wq, wk, wv)


SCALE = 1.0 / (D ** 0.5)


def _k8a_body(hp_ref, qp_ref, kl_ref, vl_ref, out_ref):
    q = qp_ref[...]
    s = lax.dot_general(q, kl_ref[...], (((1,), (1,)), ((), ())),
                        preferred_element_type=f32) * SCALE
    col = lax.broadcasted_iota(jnp.int32, s.shape, 1)
    s = jnp.where(col < NL, s, -1e30)
    m = jnp.max(s, axis=1, keepdims=True)
    p = jnp.exp(s - m)
    l = jnp.sum(p, axis=1, keepdims=True)
    out_ref[...] = hp_ref[...] + jnp.dot(p, vl_ref[...],
                                         preferred_element_type=f32) / l


def _tc_attn_p(hp, qp, kl, vl):
    grid = NPA // BP
    return pl.pallas_call(
        _k8a_body,
        grid=(grid,),
        in_specs=[
            pl.BlockSpec((BP, D), lambda i: (i, 0)),
            pl.BlockSpec((BP, D), lambda i: (i, 0)),
            pl.BlockSpec((NLA, D), lambda i: (0, 0)),
            pl.BlockSpec((NLA, D), lambda i: (0, 0)),
        ],
        out_specs=pl.BlockSpec((BP, D), lambda i: (i, 0)),
        out_shape=jax.ShapeDtypeStruct((NPA, D), f32),
    )(hp, qp, kl, vl)


def _k8b_body(hl_ref, ql_ref, kp_ref, vp_ref, out_ref, acc, mref, lref):
    k = pl.program_id(0)
    nk = pl.num_programs(0)

    @pl.when(k == 0)
    def _init():
        acc[...] = jnp.zeros_like(acc)
        mref[...] = jnp.full_like(mref, -1e30)
        lref[...] = jnp.zeros_like(lref)

    q = ql_ref[...]
    s = lax.dot_general(q, kp_ref[...], (((1,), (1,)), ((), ())),
                        preferred_element_type=f32) * SCALE
    col = lax.broadcasted_iota(jnp.int32, s.shape, 1) + k * BP
    s = jnp.where(col < NP, s, -1e30)
    m_prev = mref[...]
    m_new = jnp.maximum(m_prev, jnp.max(s, axis=1, keepdims=True))
    p = jnp.exp(s - m_new)
    alpha = jnp.exp(m_prev - m_new)
    mref[...] = m_new
    lref[...] = lref[...] * alpha + jnp.sum(p, axis=1, keepdims=True)
    acc[...] = acc[...] * alpha + jnp.dot(p, vp_ref[...],
                                          preferred_element_type=f32)

    @pl.when(k == nk - 1)
    def _fin():
        out_ref[...] = hl_ref[...] + acc[...] / lref[...]


def _tc_attn_l(hl, ql, kp, vp):
    grid = NPA // BP
    return pl.pallas_call(
        _k8b_body,
        grid=(grid,),
        in_specs=[
            pl.BlockSpec((NLA, D), lambda i: (0, 0)),
            pl.BlockSpec((NLA, D), lambda i: (0, 0)),
            pl.BlockSpec((BP, D), lambda i: (i, 0)),
            pl.BlockSpec((BP, D), lambda i: (i, 0)),
        ],
        out_specs=pl.BlockSpec((NLA, D), lambda i: (0, 0)),
        out_shape=jax.ShapeDtypeStruct((NLA, D), f32),
        scratch_shapes=[pltpu.VMEM((NLA, D), f32),
                        pltpu.VMEM((NLA, 1), f32),
                        pltpu.VMEM((NLA, 1), f32)],
    )(hl, ql, kp, vp)


def kernel(x_protein, coords_protein, edge_index_protein, x_ligand,
           edge_index_ligand, We1, be1, We2, be2, Wc, Wn1, bn1, Wn2, bn2,
           eps, Wg1, bg1, Wg2, bg2, Wq, Wk, Wv):
    h_p = jnp.zeros((NPA, D), f32).at[:NP].set(x_protein)
    c16 = jnp.zeros((NPA, 16), f32).at[:NP, :3].set(coords_protein)
    h_l = jnp.zeros((NLA, D), f32).at[:NL].set(x_ligand)

    srcp = jnp.full((EPA,), NP, jnp.int32).at[:EP].set(edge_index_protein[0])
    dstp = jnp.full((EPA,), NP, jnp.int32).at[:EP].set(edge_index_protein[1])
    srcp = srcp.reshape(EPA // CHUNK, CHUNK)
    dstp = dstp.reshape(EPA // CHUNK, CHUNK)
    srcl = jnp.full((ELA,), NLA - 1, jnp.int32).at[:EL].set(edge_index_ligand[0])
    dstl = jnp.full((ELA,), NLA - 1, jnp.int32).at[:EL].set(edge_index_ligand[1])
    srcl = srcl.reshape(ELA // CHUNK, CHUNK)
    dstl = dstl.reshape(ELA // CHUNK, CHUNK)

    zeros_p = jnp.zeros((NPA, W), f32)
    zeros_l = jnp.zeros((NLA, D), f32)

    for i in range(LAYERS):
        wa = We1[i, :D]
        wb = We1[i, D:2 * D]
        wce = We1[i, 2 * D:2 * D + 1]
        be1_i = be1[i:i + 1]
        be2_i = be2[i:i + 1]
        wc_i = Wc[i].T
        wn1a = Wn1[i, :D]
        wn1b = Wn1[i, D:]
        bn1_i = bn1[i:i + 1]
        bn2_i = bn2[i:i + 1]
        eps_i = eps[i:i + 1]
        bg1_i = bg1[i:i + 1]
        bg2_i = bg2[i:i + 1]

        a2 = _sc_gin_agg(h_l, srcl, dstl, zeros_l)
        rd, rs = _tc_node_project(h_p, c16, wa, wb, be1_i)
        hrows = EPA // CHUNK // 2
        half = EPA // 2
        ed, es = _sc_gather(rd, rs, dstp, srcp)
        em0 = _tc_edge_mlp(ed, es, wce, We2[i], be2_i, wc_i, 0, half)
        sa = _sc_scatter(em0, dstp[:hrows], zeros_p)
        em1 = _tc_edge_mlp(ed, es, wce, We2[i], be2_i, wc_i, half, half)
        sb = _sc_scatter(em1, dstp[hrows:], zeros_p)
        h_p, c16, qp, kp, vp = _tc_node_update(h_p, c16, sa, sb,
                                               wn1a, wn1b, bn1_i, Wn2[i],
                                               bn2_i, Wq[i], Wk[i], Wv[i])
        h_l, ql, kl, vl = _tc_gin_update(h_l, a2, eps_i,
                                         Wg1[i], bg1_i, Wg2[i], bg2_i,
                                         Wq[i], Wk[i], Wv[i])
        h_p_new = _tc_attn_p(h_p, qp, kl, vl)
        h_l = _tc_attn_l(h_l, ql, kp, vp)
        h_p = h_p_new

    return h_p[:NP], h_l[:NL]


# speedup vs baseline: 1.0386x; 1.0386x over previous
import functools

import jax
import jax.numpy as jnp
from jax import lax
from jax.experimental import pallas as pl
from jax.experimental.pallas import tpu as pltpu
from jax.experimental.pallas import tpu_sc as plsc

NP = 10000
EP = 160000
NL = 2000
EL = 32000
D = 128
LAYERS = 3

NPA = 10240
NLA = 2048
EPA = 163840
ELA = 32768
W = 144

NC, NS = 2, 16
NW = NC * NS
CHUNK = 128
EP_CHUNKS = EPA // (NW * CHUNK)
EL_CHUNKS = ELA // (NW * CHUNK)
ROWS_P = NPA // NS
ROWS_L = NLA // NS

f32 = jnp.float32
bf16 = jnp.bfloat16


@functools.lru_cache(maxsize=None)
def _sc_mesh():
    return plsc.VectorSubcoreMesh(core_axis_name="c", subcore_axis_name="s",
                                  num_cores=NC, num_subcores=NS)


def _make_gather_body(cpw):
    def body_fn(rd_hbm, rs_hbm, dsti_hbm, srci_hbm, ed_hbm, es_hbm,
                idx_da, idx_sa, buf_d0, buf_s0, buf_d1, buf_s1,
                gsem0, gsem1, wsem0, wsem1):
        wid = lax.axis_index("s") * NC + lax.axis_index("c")
        base = wid * cpw
        bufs_d = (buf_d0, buf_d1)
        bufs_s = (buf_s0, buf_s1)
        gsems = (gsem0, gsem1)
        wsems = (wsem0, wsem1)
        pltpu.sync_copy(dsti_hbm.at[pl.ds(base, cpw)], idx_da)
        pltpu.sync_copy(srci_hbm.at[pl.ds(base, cpw)], idx_sa)

        def gstart(j, b):
            pltpu.async_copy(rd_hbm.at[idx_da.at[j]], bufs_d[b], gsems[b])
            pltpu.async_copy(rs_hbm.at[idx_sa.at[j]], bufs_s[b], gsems[b])

        def gwait(b):
            pltpu.make_async_copy(rd_hbm.at[idx_da.at[0]], bufs_d[b],
                                  gsems[b]).wait()
            pltpu.make_async_copy(rs_hbm.at[idx_sa.at[0]], bufs_s[b],
                                  gsems[b]).wait()

        def wstart(j, b):
            r = base + j
            pltpu.async_copy(bufs_d[b], ed_hbm.at[pl.ds(r * CHUNK, CHUNK)],
                             wsems[b])
            pltpu.async_copy(bufs_s[b], es_hbm.at[pl.ds(r * CHUNK, CHUNK)],
                             wsems[b])

        def wwait(b):
            pltpu.make_async_copy(bufs_d[b], ed_hbm.at[pl.ds(0, CHUNK)],
                                  wsems[b]).wait()
            pltpu.make_async_copy(bufs_s[b], es_hbm.at[pl.ds(0, CHUNK)],
                                  wsems[b]).wait()

        gstart(0, 0)
        gstart(1, 1)

        def outer(t, carry):
            for b in (0, 1):
                j = 2 * t + b
                gwait(b)
                wstart(j, b)
                wwait(b)
                gstart(j + 2, b)
            return carry

        lax.fori_loop(0, cpw // 2 - 1, outer, 0)
        for b in (0, 1):
            gwait(b)
            wstart(cpw - 2 + b, b)
            wwait(b)

    return body_fn


@functools.lru_cache(maxsize=None)
def _sc_gather_call(n_edges):
    cpw = n_edges // (NW * CHUNK)
    return pl.kernel(
        _make_gather_body(cpw),
        out_type=[jax.ShapeDtypeStruct((n_edges, W), f32),
                  jax.ShapeDtypeStruct((n_edges, W), f32)],
        mesh=_sc_mesh(),
        compiler_params=pltpu.CompilerParams(use_tc_tiling_on_sc=False),
        scratch_types=[
            pltpu.VMEM((cpw, CHUNK), jnp.int32),
            pltpu.VMEM((cpw, CHUNK), jnp.int32),
            pltpu.VMEM((CHUNK, W), f32),
            pltpu.VMEM((CHUNK, W), f32),
            pltpu.VMEM((CHUNK, W), f32),
            pltpu.VMEM((CHUNK, W), f32),
            pltpu.SemaphoreType.DMA,
            pltpu.SemaphoreType.DMA,
            pltpu.SemaphoreType.DMA,
            pltpu.SemaphoreType.DMA,
        ],
    )


def _sc_gather(rd, rs, dstp, srcp):
    return _sc_gather_call(dstp.shape[0] * CHUNK)(rd, rs, dstp, srcp)


def _make_scatter_body(cpw):
    def body_fn(em_hbm, dsti_hbm, zeros_hbm, out_hbm, idx0, idx1,
                buf0, buf1, acc, lsem0, lsem1, ssem0, ssem1):
        c = lax.axis_index("c")
        s = lax.axis_index("s")
        wid = s * NC + c
        base = wid * cpw
        idxs = (idx0, idx1)
        bufs = (buf0, buf1)
        lsems = (lsem0, lsem1)
        ssems = (ssem0, ssem1)
        pltpu.sync_copy(zeros_hbm.at[pl.ds(s * ROWS_P, ROWS_P)],
                        acc.at[pl.ds(s * ROWS_P, ROWS_P)])
        plsc.subcore_barrier()

        def lstart(j, b):
            r = base + j
            pltpu.async_copy(dsti_hbm.at[r], idxs[b], lsems[b])
            pltpu.async_copy(em_hbm.at[pl.ds(r * CHUNK, CHUNK)], bufs[b],
                             lsems[b])

        def lwait(b):
            pltpu.make_async_copy(dsti_hbm.at[0], idxs[b], lsems[b]).wait()
            pltpu.make_async_copy(em_hbm.at[pl.ds(0, CHUNK)], bufs[b],
                                  lsems[b]).wait()

        def sstart(j, b):
            pltpu.async_copy(bufs[b], acc.at[idxs[b]], ssems[b], add=True)

        def swait(b):
            pltpu.make_async_copy(bufs[b], acc.at[idxs[b]], ssems[b]).wait()

        lstart(0, 0)
        lstart(1, 1)

        def outer(t, carry):
            for b in (0, 1):
                j = 2 * t + b
                lwait(b)
                sstart(j, b)
                swait(b)
                lstart(j + 2, b)
            return carry

        lax.fori_loop(0, cpw // 2 - 1, outer, 0)
        for b in (0, 1):
            lwait(b)
            sstart(cpw - 2 + b, b)
            swait(b)
        plsc.subcore_barrier()
        pltpu.sync_copy(acc.at[pl.ds(s * ROWS_P, ROWS_P)],
                        out_hbm.at[c, pl.ds(s * ROWS_P, ROWS_P)])

    return body_fn


@functools.lru_cache(maxsize=None)
def _sc_scatter_call(n_edges):
    cpw = n_edges // (NW * CHUNK)
    return pl.kernel(
        _make_scatter_body(cpw),
        out_type=jax.ShapeDtypeStruct((NC, NPA, W), f32),
        mesh=_sc_mesh(),
        compiler_params=pltpu.CompilerParams(use_tc_tiling_on_sc=False),
        scratch_types=[
            pltpu.VMEM((CHUNK,), jnp.int32),
            pltpu.VMEM((CHUNK,), jnp.int32),
            pltpu.VMEM((CHUNK, W), f32),
            pltpu.VMEM((CHUNK, W), f32),
            pltpu.VMEM_SHARED((NPA, W), f32),
            pltpu.SemaphoreType.DMA,
            pltpu.SemaphoreType.DMA,
            pltpu.SemaphoreType.DMA,
            pltpu.SemaphoreType.DMA,
        ],
    )


def _sc_scatter(em, dstp, zeros_p):
    return _sc_scatter_call(dstp.shape[0] * CHUNK)(em, dstp, zeros_p)


def _gin_body(hl_hbm, srci_hbm, dsti_hbm, zeros_hbm, out_hbm,
              idx_sa, idx_da, buf0, buf1, acc, gsem0, gsem1, ssem0, ssem1):
    c = lax.axis_index("c")
    s = lax.axis_index("s")
    wid = s * NC + c
    base = wid * EL_CHUNKS
    bufs = (buf0, buf1)
    gsems = (gsem0, gsem1)
    ssems = (ssem0, ssem1)
    pltpu.sync_copy(zeros_hbm.at[pl.ds(s * ROWS_L, ROWS_L)],
                    acc.at[pl.ds(s * ROWS_L, ROWS_L)])
    pltpu.sync_copy(srci_hbm.at[pl.ds(base, EL_CHUNKS)], idx_sa)
    pltpu.sync_copy(dsti_hbm.at[pl.ds(base, EL_CHUNKS)], idx_da)
    plsc.subcore_barrier()

    def gstart(j, b):
        pltpu.async_copy(hl_hbm.at[idx_sa.at[j]], bufs[b], gsems[b])

    def gwait(b):
        pltpu.make_async_copy(hl_hbm.at[idx_sa.at[0]], bufs[b],
                              gsems[b]).wait()

    def sstart(j, b):
        pltpu.async_copy(bufs[b], acc.at[idx_da.at[j]], ssems[b], add=True)

    def swait(b):
        pltpu.make_async_copy(bufs[b], acc.at[idx_da.at[0]], ssems[b]).wait()

    gstart(0, 0)
    gstart(1, 1)

    def outer(t, carry):
        for b in (0, 1):
            j = 2 * t + b
            gwait(b)
            sstart(j, b)
            swait(b)
            gstart(j + 2, b)
        return carry

    lax.fori_loop(0, EL_CHUNKS // 2 - 1, outer, 0)
    for b in (0, 1):
        gwait(b)
        sstart(EL_CHUNKS - 2 + b, b)
        swait(b)
    plsc.subcore_barrier()
    pltpu.sync_copy(acc.at[pl.ds(s * ROWS_L, ROWS_L)],
                    out_hbm.at[c, pl.ds(s * ROWS_L, ROWS_L)])


@functools.lru_cache(maxsize=None)
def _sc_gin_agg_call():
    return pl.kernel(
        _gin_body,
        out_type=jax.ShapeDtypeStruct((NC, NLA, D), f32),
        mesh=_sc_mesh(),
        compiler_params=pltpu.CompilerParams(use_tc_tiling_on_sc=False),
        scratch_types=[
            pltpu.VMEM((EL_CHUNKS, CHUNK), jnp.int32),
            pltpu.VMEM((EL_CHUNKS, CHUNK), jnp.int32),
            pltpu.VMEM((CHUNK, D), f32),
            pltpu.VMEM((CHUNK, D), f32),
            pltpu.VMEM_SHARED((NLA, D), f32),
            pltpu.SemaphoreType.DMA,
            pltpu.SemaphoreType.DMA,
            pltpu.SemaphoreType.DMA,
            pltpu.SemaphoreType.DMA,
        ],
    )


def _sc_gin_agg(h_l, srcl, dstl, zeros_l):
    return _sc_gin_agg_call()(h_l, srcl, dstl, zeros_l)


BP = 512
BE = 1024


def _k1_body(h_ref, c_ref, wa_ref, wb_ref, be1_ref, rd_ref, rs_ref):
    h = h_ref[...]
    a = jnp.dot(h, wa_ref[...], preferred_element_type=f32) + be1_ref[...]
    b = jnp.dot(h, wb_ref[...], preferred_element_type=f32)
    cpad = c_ref[...]
    rd_ref[:, :D] = a
    rd_ref[:, D:] = cpad
    rs_ref[:, :D] = b
    rs_ref[:, D:] = cpad


def _tc_node_project(h, c16, wa, wb, be1):
    grid = NPA // BP
    return pl.pallas_call(
        _k1_body,
        grid=(grid,),
        in_specs=[
            pl.BlockSpec((BP, D), lambda i: (i, 0)),
            pl.BlockSpec((BP, 16), lambda i: (i, 0)),
            pl.BlockSpec((D, D), lambda i: (0, 0)),
            pl.BlockSpec((D, D), lambda i: (0, 0)),
            pl.BlockSpec((1, D), lambda i: (0, 0)),
        ],
        out_specs=[pl.BlockSpec((BP, W), lambda i: (i, 0)),
                   pl.BlockSpec((BP, W), lambda i: (i, 0))],
        out_shape=[jax.ShapeDtypeStruct((NPA, W), f32),
                   jax.ShapeDtypeStruct((NPA, W), f32)],
    )(h, c16, wa, wb, be1)


def _k3_body(ed_ref, es_ref, wce_ref, we2_ref, be2_ref, wc_ref, em_ref):
    ed = ed_ref[...]
    es = es_ref[...]
    a = ed[:, :D]
    b = es[:, :D]
    rel = ed[:, D:] - es[:, D:]
    d2 = jnp.sum(rel * rel, axis=1, keepdims=True)
    pre = a + b + d2 * wce_ref[...]
    m1 = jax.nn.silu(pre)
    m2 = jax.nn.silu(jnp.dot(m1, we2_ref[...], preferred_element_type=f32)
                     + be2_ref[...])
    cw = jnp.sum(m2 * wc_ref[...], axis=1, keepdims=True)
    relcw = rel * cw
    lane = lax.broadcasted_iota(jnp.int32, relcw.shape, 1)
    relcw = jnp.where(lane == 3, 1.0, relcw)
    em_ref[:, :D] = m2
    em_ref[:, D:] = relcw


def _tc_edge_mlp(ed, es, wce, we2, be2, wc, off, n):
    grid = n // BE
    o = off // BE
    return pl.pallas_call(
        _k3_body,
        grid=(grid,),
        in_specs=[
            pl.BlockSpec((BE, W), lambda i: (i + o, 0)),
            pl.BlockSpec((BE, W), lambda i: (i + o, 0)),
            pl.BlockSpec((1, D), lambda i: (0, 0)),
            pl.BlockSpec((D, D), lambda i: (0, 0)),
            pl.BlockSpec((1, D), lambda i: (0, 0)),
            pl.BlockSpec((1, D), lambda i: (0, 0)),
        ],
        out_specs=pl.BlockSpec((BE, W), lambda i: (i, 0)),
        out_shape=jax.ShapeDtypeStruct((n, W), f32),
    )(ed, es, wce, we2, be2, wc)


def _k5_body(h_ref, c_ref, sa_ref, sb_ref,
             wn1a_ref, wn1b_ref, bn1_ref,
             wn2_ref, bn2_ref, wq_ref, wk_ref, wv_ref,
             hout_ref, cout_ref, q_ref, k_ref, v_ref):
    S = (sa_ref[0] + sa_ref[1]) + (sb_ref[0] + sb_ref[1])
    agg = S[:, :D]
    sc = S[:, D:]
    deg = jnp.maximum(sc[:, 3:4], 1.0)
    lane = lax.broadcasted_iota(jnp.int32, sc.shape, 1)
    cout_ref[...] = c_ref[...] + jnp.where(lane < 3, sc / deg, 0.0)
    h = h_ref[...]
    t = jax.nn.silu(jnp.dot(h, wn1a_ref[...], preferred_element_type=f32)
                    + jnp.dot(agg, wn1b_ref[...], preferred_element_type=f32)
                    + bn1_ref[...])
    hn = h + jnp.dot(t, wn2_ref[...], preferred_element_type=f32) + bn2_ref[...]
    hout_ref[...] = hn
    q_ref[...] = jnp.dot(hn, wq_ref[...], preferred_element_type=f32)
    k_ref[...] = jnp.dot(hn, wk_ref[...], preferred_element_type=f32)
    v_ref[...] = jnp.dot(hn, wv_ref[...], preferred_element_type=f32)


def _tc_node_update(h, c16, sa, sb, wn1a, wn1b, bn1, wn2, bn2,
                    wq, wk, wv):
    grid = NPA // BP
    wspec = pl.BlockSpec((D, D), lambda i: (0, 0))
    rowspec = pl.BlockSpec((BP, D), lambda i: (i, 0))
    return pl.pallas_call(
        _k5_body,
        grid=(grid,),
        in_specs=[
            rowspec,
            pl.BlockSpec((BP, 16), lambda i: (i, 0)),
            pl.BlockSpec((NC, BP, W), lambda i: (0, i, 0)),
            pl.BlockSpec((NC, BP, W), lambda i: (0, i, 0)),
            wspec,
            wspec,
            pl.BlockSpec((1, D), lambda i: (0, 0)),
            wspec,
            pl.BlockSpec((1, D), lambda i: (0, 0)),
            wspec,
            wspec,
            wspec,
        ],
        out_specs=[rowspec,
                   pl.BlockSpec((BP, 16), lambda i: (i, 0)),
                   rowspec, rowspec, rowspec],
        out_shape=[jax.ShapeDtypeStruct((NPA, D), f32),
                   jax.ShapeDtypeStruct((NPA, 16), f32),
                   jax.ShapeDtypeStruct((NPA, D), f32),
                   jax.ShapeDtypeStruct((NPA, D), f32),
                   jax.ShapeDtypeStruct((NPA, D), f32)],
    )(h, c16, sa, sb, wn1a, wn1b, bn1, wn2, bn2, wq, wk, wv)


def _k7_body(h_ref, a2_ref, eps_ref, wg1_ref, bg1_ref, wg2_ref,
             bg2_ref, wq_ref, wk_ref, wv_ref, out_ref, q_ref, k_ref, v_ref):
    z = (1.0 + eps_ref[0]) * h_ref[...] + a2_ref[0] + a2_ref[1]
    h1 = jax.nn.relu(jnp.dot(z, wg1_ref[...], preferred_element_type=f32)
                     + bg1_ref[...])
    hn = jnp.dot(h1, wg2_ref[...], preferred_element_type=f32) + bg2_ref[...]
    out_ref[...] = hn
    q_ref[...] = jnp.dot(hn, wq_ref[...], preferred_element_type=f32)
    k_ref[...] = jnp.dot(hn, wk_ref[...], preferred_element_type=f32)
    v_ref[...] = jnp.dot(hn, wv_ref[...], preferred_element_type=f32)


def _tc_gin_update(h, a2, eps_i, wg1, bg1, wg2, bg2, wq, wk, wv):
    grid = NLA // BP
    wspec = pl.BlockSpec((D, D), lambda i: (0, 0))
    rowspec = pl.BlockSpec((BP, D), lambda i: (i, 0))
    return pl.pallas_call(
        _k7_body,
        grid=(grid,),
        in_specs=[
            rowspec,
            pl.BlockSpec((NC, BP, D), lambda i: (0, i, 0)),
            pl.BlockSpec(memory_space=pltpu.SMEM),
            wspec,
            pl.BlockSpec((1, D), lambda i: (0, 0)),
            wspec,
            pl.BlockSpec((1, D), lambda i: (0, 0)),
            wspec,
            wspec,
            wspec,
        ],
        out_specs=[rowspec] * 4,
        out_shape=[jax.ShapeDtypeStruct((NLA, D), f32)] * 4,
    )(h, a2, eps_i, wg1, bg1, wg2, bg2, wq, wk, wv)


SCALE = 1.0 / (D ** 0.5)


def _k8a_body(hp_ref, qp_ref, kl_ref, vl_ref, out_ref):
    q = qp_ref[...]
    s = lax.dot_general(q, kl_ref[...], (((1,), (1,)), ((), ())),
                        preferred_element_type=f32) * SCALE
    col = lax.broadcasted_iota(jnp.int32, s.shape, 1)
    s = jnp.where(col < NL, s, -1e30)
    m = jnp.max(s, axis=1, keepdims=True)
    p = jnp.exp(s - m)
    l = jnp.sum(p, axis=1, keepdims=True)
    out_ref[...] = hp_ref[...] + jnp.dot(p, vl_ref[...],
                                         preferred_element_type=f32) / l


def _tc_attn_p(hp, qp, kl, vl):
    grid = NPA // BP
    return pl.pallas_call(
        _k8a_body,
        grid=(grid,),
        in_specs=[
            pl.BlockSpec((BP, D), lambda i: (i, 0)),
            pl.BlockSpec((BP, D), lambda i: (i, 0)),
            pl.BlockSpec((NLA, D), lambda i: (0, 0)),
            pl.BlockSpec((NLA, D), lambda i: (0, 0)),
        ],
        out_specs=pl.BlockSpec((BP, D), lambda i: (i, 0)),
        out_shape=jax.ShapeDtypeStruct((NPA, D), f32),
    )(hp, qp, kl, vl)


def _k8b_body(hl_ref, ql_ref, kp_ref, vp_ref, out_ref, acc, mref, lref):
    k = pl.program_id(0)
    nk = pl.num_programs(0)

    @pl.when(k == 0)
    def _init():
        acc[...] = jnp.zeros_like(acc)
        mref[...] = jnp.full_like(mref, -1e30)
        lref[...] = jnp.zeros_like(lref)

    q = ql_ref[...]
    s = lax.dot_general(q, kp_ref[...], (((1,), (1,)), ((), ())),
                        preferred_element_type=f32) * SCALE
    col = lax.broadcasted_iota(jnp.int32, s.shape, 1) + k * BP
    s = jnp.where(col < NP, s, -1e30)
    m_prev = mref[...]
    m_new = jnp.maximum(m_prev, jnp.max(s, axis=1, keepdims=True))
    p = jnp.exp(s - m_new)
    alpha = jnp.exp(m_prev - m_new)
    mref[...] = m_new
    lref[...] = lref[...] * alpha + jnp.sum(p, axis=1, keepdims=True)
    acc[...] = acc[...] * alpha + jnp.dot(p, vp_ref[...],
                                          preferred_element_type=f32)

    @pl.when(k == nk - 1)
    def _fin():
        out_ref[...] = hl_ref[...] + acc[...] / lref[...]


def _tc_attn_l(hl, ql, kp, vp):
    grid = NPA // BP
    return pl.pallas_call(
        _k8b_body,
        grid=(grid,),
        in_specs=[
            pl.BlockSpec((NLA, D), lambda i: (0, 0)),
            pl.BlockSpec((NLA, D), lambda i: (0, 0)),
            pl.BlockSpec((BP, D), lambda i: (i, 0)),
            pl.BlockSpec((BP, D), lambda i: (i, 0)),
        ],
        out_specs=pl.BlockSpec((NLA, D), lambda i: (0, 0)),
        out_shape=jax.ShapeDtypeStruct((NLA, D), f32),
        scratch_shapes=[pltpu.VMEM((NLA, D), f32),
                        pltpu.VMEM((NLA, 1), f32),
                        pltpu.VMEM((NLA, 1), f32)],
    )(hl, ql, kp, vp)


def kernel(x_protein, coords_protein, edge_index_protein, x_ligand,
           edge_index_ligand, We1, be1, We2, be2, Wc, Wn1, bn1, Wn2, bn2,
           eps, Wg1, bg1, Wg2, bg2, Wq, Wk, Wv):
    h_p = jnp.zeros((NPA, D), f32).at[:NP].set(x_protein)
    c16 = jnp.zeros((NPA, 16), f32).at[:NP, :3].set(coords_protein)
    h_l = jnp.zeros((NLA, D), f32).at[:NL].set(x_ligand)

    srcp = jnp.full((EPA,), NP, jnp.int32).at[:EP].set(edge_index_protein[0])
    dstp = jnp.full((EPA,), NP, jnp.int32).at[:EP].set(edge_index_protein[1])
    srcp = srcp.reshape(EPA // CHUNK, CHUNK)
    dstp = dstp.reshape(EPA // CHUNK, CHUNK)
    srcl = jnp.full((ELA,), NLA - 1, jnp.int32).at[:EL].set(edge_index_ligand[0])
    dstl = jnp.full((ELA,), NLA - 1, jnp.int32).at[:EL].set(edge_index_ligand[1])
    srcl = srcl.reshape(ELA // CHUNK, CHUNK)
    dstl = dstl.reshape(ELA // CHUNK, CHUNK)

    zeros_p = jnp.zeros((NPA, W), f32)
    zeros_l = jnp.zeros((NLA, D), f32)

    for i in range(LAYERS):
        wa = We1[i, :D]
        wb = We1[i, D:2 * D]
        wce = We1[i, 2 * D:2 * D + 1]
        be1_i = be1[i:i + 1]
        be2_i = be2[i:i + 1]
        wc_i = Wc[i].T
        wn1a = Wn1[i, :D]
        wn1b = Wn1[i, D:]
        bn1_i = bn1[i:i + 1]
        bn2_i = bn2[i:i + 1]
        eps_i = eps[i:i + 1]
        bg1_i = bg1[i:i + 1]
        bg2_i = bg2[i:i + 1]

        a2 = _sc_gin_agg(h_l, srcl, dstl, zeros_l)
        rd, rs = _tc_node_project(h_p, c16, wa, wb, be1_i)
        hrows = EPA // CHUNK // 2
        half = EPA // 2
        ed0, es0 = _sc_gather(rd, rs, dstp[:hrows], srcp[:hrows])
        em0 = _tc_edge_mlp(ed0, es0, wce, We2[i], be2_i, wc_i, 0, half)
        ed1, es1 = _sc_gather(rd, rs, dstp[hrows:], srcp[hrows:])
        sa = _sc_scatter(em0, dstp[:hrows], zeros_p)
        em1 = _tc_edge_mlp(ed1, es1, wce, We2[i], be2_i, wc_i, 0, half)
        sb = _sc_scatter(em1, dstp[hrows:], zeros_p)
        h_p, c16, qp, kp, vp = _tc_node_update(h_p, c16, sa, sb,
                                               wn1a, wn1b, bn1_i, Wn2[i],
                                               bn2_i, Wq[i], Wk[i], Wv[i])
        h_l, ql, kl, vl = _tc_gin_update(h_l, a2, eps_i,
                                         Wg1[i], bg1_i, Wg2[i], bg2_i,
                                         Wq[i], Wk[i], Wv[i])
        h_p_new = _tc_attn_p(h_p, qp, kl, vl)
        h_l = _tc_attn_l(h_l, ql, kp, vp)
        h_p = h_p_new

    return h_p[:NP], h_l[:NL]
